# Initial kernel scaffold; baseline (speedup 1.0000x reference)
import jax
import jax.numpy as jnp
from jax.experimental import pallas as pl


def kernel(ind_2, px, Wi, Wj, Wff):
    raise NotImplementedError("write your pallas kernel here")


# speedup vs baseline: 87.0647x; 87.0647x over previous
import functools

import jax
import jax.numpy as jnp
from jax import lax
from jax.experimental import pallas as pl
from jax.experimental.pallas import tpu as pltpu
from jax.experimental.pallas import tpu_sc as plsc

D = 16
X = 3
CHUNK = 1024
IDXW = 128
ZROWS = 896


def _sc_edge_kernel(n_pad, e_pad):
    n_tiles = 16
    rows_per_tile = n_pad // n_tiles
    ept = e_pad // (2 * n_tiles)
    idx_rows_per_tile = ept // IDXW
    n_chunks = ept // CHUNK
    assert n_pad % (n_tiles * ZROWS) == 0
    assert ept % CHUNK == 0 and CHUNK % IDXW == 0

    mesh = plsc.VectorSubcoreMesh(core_axis_name="c", subcore_axis_name="s")

    @functools.partial(
        pl.kernel,
        out_type=jax.ShapeDtypeStruct((2, 4, n_pad, D), jnp.float32),
        mesh=mesh,
        compiler_params=pltpu.CompilerParams(use_tc_tiling_on_sc=False),
        scratch_types=[
            pltpu.VMEM((CHUNK // IDXW, IDXW), jnp.int32),
            pltpu.VMEM((CHUNK // IDXW, IDXW), jnp.int32),
            pltpu.VMEM((CHUNK, D), jnp.float32),
            pltpu.VMEM_SHARED((n_pad, D), jnp.float32),
            pltpu.SemaphoreType.DMA,
        ],
    )
    def body(ii2, jj2, px0, px1, px2, sp, ib, jb, rows, acc, sem):
        c = lax.axis_index("c")
        t = lax.axis_index("s")
        tables = (px0, px1, px2)

        def fill_rows(count, value):
            row = jnp.full((D,), value, dtype=jnp.float32)

            def fill(i, _):
                rows[i, :] = row
                return 0

            lax.fori_loop(0, count, fill, 0)

        tile_row0 = t * rows_per_tile
        idx_row0 = (c * n_tiles + t) * idx_rows_per_tile

        for p in range(4):
            fill_rows(ZROWS, 0.0)
            zsrc = rows.at[pl.ds(0, ZROWS), :]
            for z in range(rows_per_tile // ZROWS):
                pltpu.sync_copy(zsrc, acc.at[pl.ds(tile_row0 + z * ZROWS, ZROWS), :])
            if p == 3:
                fill_rows(IDXW, 1.0)
            plsc.subcore_barrier()

            def chunk_body(k, _):
                r0 = idx_row0 + k * (CHUNK // IDXW)
                pltpu.sync_copy(ii2.at[pl.ds(r0, CHUNK // IDXW)], ib)
                if p < 3:
                    pltpu.sync_copy(jj2.at[pl.ds(r0, CHUNK // IDXW)], jb)
                for r in range(CHUNK // IDXW):
                    if p < 3:
                        dst = rows.at[pl.ds(r * IDXW, IDXW), :]
                        pltpu.async_copy(tables[p].at[jb.at[r]], dst, sem).wait()
                        pltpu.sync_copy(dst, acc.at[ib.at[r]], add=True)
                    else:
                        ones = rows.at[pl.ds(0, IDXW), :]
                        pltpu.sync_copy(ones, acc.at[ib.at[r]], add=True)
                return 0

            lax.fori_loop(0, n_chunks, chunk_body, 0)
            plsc.subcore_barrier()

            for z in range(rows_per_tile // ZROWS):
                r0 = tile_row0 + z * ZROWS
                stage = rows.at[pl.ds(0, ZROWS), :]
                pltpu.sync_copy(acc.at[pl.ds(r0, ZROWS), :], stage)
                pltpu.sync_copy(stage, sp.at[c, p, pl.ds(r0, ZROWS), :])

    return body


def _tc_combine(px, sp, Wi, Wj, Wff, block_n):
    n = px.shape[0]
    assert n % block_n == 0

    def body(px_ref, sp_ref, wi_ref, wj_ref, wff_ref, out_ref):
        a = jnp.dot(wi_ref[...], wff_ref[...], preferred_element_type=jnp.float32)
        b = jnp.dot(wj_ref[...], wff_ref[...], preferred_element_type=jnp.float32)
        deg = sp_ref[0, 3] + sp_ref[1, 3]
        for s in range(X):
            x = px_ref[:, s, :]
            ssum = sp_ref[0, s] + sp_ref[1, s]
            out_ref[:, s, :] = deg * jnp.dot(x, a, preferred_element_type=jnp.float32) \
                + jnp.dot(ssum, b, preferred_element_type=jnp.float32)

    return pl.pallas_call(
        body,
        grid=(n // block_n,),
        in_specs=[
            pl.BlockSpec((block_n, X, D), lambda i: (i, 0, 0)),
            pl.BlockSpec((2, 4, block_n, D), lambda i: (0, 0, i, 0)),
            pl.BlockSpec((D, D), lambda i: (0, 0)),
            pl.BlockSpec((D, D), lambda i: (0, 0)),
            pl.BlockSpec((D, D), lambda i: (0, 0)),
        ],
        out_specs=pl.BlockSpec((block_n, X, D), lambda i: (i, 0, 0)),
        out_shape=jax.ShapeDtypeStruct((n, X, D), jnp.float32),
    )(px, sp, Wi, Wj, Wff)


def kernel(ind_2, px, Wi, Wj, Wff):
    e = ind_2.shape[0]
    n = px.shape[0]

    ept = -(-e // (32 * CHUNK)) * CHUNK
    e_pad = 32 * ept
    n_pad = -(-(n + 1) // (16 * ZROWS)) * (16 * ZROWS)

    ii = ind_2[:, 0]
    jj = ind_2[:, 1]
    ii = jnp.concatenate([ii, jnp.full((e_pad - e,), n, dtype=jnp.int32)])
    jj = jnp.concatenate([jj, jnp.zeros((e_pad - e,), dtype=jnp.int32)])
    ii2 = ii.reshape(e_pad // IDXW, IDXW)
    jj2 = jj.reshape(e_pad // IDXW, IDXW)

    px_t = jnp.transpose(px, (1, 0, 2))
    sc = _sc_edge_kernel(n_pad, e_pad)
    sp = sc(ii2, jj2, px_t[0], px_t[1], px_t[2])

    return _tc_combine(px, sp, Wi, Wj, Wff, block_n=1000)


# speedup vs baseline: 124.1995x; 1.4265x over previous
import functools

import jax
import jax.numpy as jnp
from jax import lax
from jax.experimental import pallas as pl
from jax.experimental.pallas import tpu as pltpu
from jax.experimental.pallas import tpu_sc as plsc

D = 16
X = 3
IDXW = 128
SUB = 4
CHUNK = SUB * IDXW
ZROWS = 448


def _sc_edge_kernel(n_pad, e_pad):
    n_tiles = 16
    rows_per_tile = n_pad // n_tiles
    ept = e_pad // (2 * n_tiles)
    n_chunks = ept // CHUNK
    n_pairs = n_chunks // 2
    assert n_pad % (n_tiles * ZROWS) == 0
    assert ept % (2 * CHUNK) == 0 and CHUNK % IDXW == 0

    mesh = plsc.VectorSubcoreMesh(core_axis_name="c", subcore_axis_name="s")

    @functools.partial(
        pl.kernel,
        out_type=jax.ShapeDtypeStruct((2, 4, n_pad, D), jnp.float32),
        mesh=mesh,
        compiler_params=pltpu.CompilerParams(use_tc_tiling_on_sc=False),
        scratch_types=[
            pltpu.VMEM((SUB, IDXW), jnp.int32),
            pltpu.VMEM((SUB, IDXW), jnp.int32),
            pltpu.VMEM((SUB, IDXW), jnp.int32),
            pltpu.VMEM((SUB, IDXW), jnp.int32),
            pltpu.VMEM((CHUNK, D), jnp.float32),
            pltpu.VMEM((CHUNK, D), jnp.float32),
            pltpu.VMEM_SHARED((n_pad, D), jnp.float32),
            pltpu.SemaphoreType.DMA,
            pltpu.SemaphoreType.DMA,
            pltpu.SemaphoreType.DMA,
        ],
    )
    def body(ii2, jj2, px0, px1, px2, sp,
             ibA, ibB, jbA, jbB, rowsA, rowsB, acc, semA, semB, semF):
        c = lax.axis_index("c")
        t = lax.axis_index("s")
        tables = (px0, px1, px2)

        def fill_rows(ref, count, value):
            row = jnp.full((D,), value, dtype=jnp.float32)

            def fill(i, _):
                ref[i, :] = row
                return 0

            lax.fori_loop(0, count, fill, 0)

        tile_row0 = t * rows_per_tile
        idx_row0 = (c * n_tiles + t) * (ept // IDXW)

        def load_idx(p, k, ib, jb):
            r0 = idx_row0 + k * SUB
            pltpu.sync_copy(ii2.at[pl.ds(r0, SUB)], ib)
            if p < 3:
                pltpu.sync_copy(jj2.at[pl.ds(r0, SUB)], jb)

        def gathers(p, jb, rows, sem):
            if p < 3:
                for r in range(SUB):
                    dst = rows.at[pl.ds(r * IDXW, IDXW), :]
                    pltpu.async_copy(tables[p].at[jb.at[r]], dst, sem)

        def scatters(p, acc, ib, rows):
            for r in range(SUB):
                if p < 3:
                    src = rows.at[pl.ds(r * IDXW, IDXW), :]
                else:
                    src = rows.at[pl.ds(0, IDXW), :]
                pltpu.sync_copy(src, acc.at[ib.at[r]], add=True)

        def run_pass(p, acc):
            fill_rows(rowsA, ZROWS, 0.0)
            zsrc = rowsA.at[pl.ds(0, ZROWS), :]
            for z in range(rows_per_tile // ZROWS):
                pltpu.sync_copy(zsrc, acc.at[pl.ds(tile_row0 + z * ZROWS, ZROWS), :])
            if p == 3:
                fill_rows(rowsA, IDXW, 1.0)
                fill_rows(rowsB, IDXW, 1.0)
            plsc.subcore_barrier()

            if p < 3:
                load_idx(p, 0, ibA, jbA)
                gathers(p, jbA, rowsA, semA)

                def pair(q, _):
                    load_idx(p, 2 * q + 1, ibB, jbB)
                    gathers(p, jbB, rowsB, semB)
                    for r in range(SUB):
                        pltpu.make_async_copy(
                            tables[p].at[jbA.at[r]],
                            rowsA.at[pl.ds(r * IDXW, IDXW), :], semA).wait()
                    scatters(p, acc, ibA, rowsA)

                    @pl.when(q < n_pairs - 1)
                    def _():
                        load_idx(p, 2 * q + 2, ibA, jbA)
                        gathers(p, jbA, rowsA, semA)

                    for r in range(SUB):
                        pltpu.make_async_copy(
                            tables[p].at[jbB.at[r]],
                            rowsB.at[pl.ds(r * IDXW, IDXW), :], semB).wait()
                    scatters(p, acc, ibB, rowsB)
                    return 0

                lax.fori_loop(0, n_pairs, pair, 0)
            else:
                load_idx(p, 0, ibA, jbA)

                def pair3(q, _):
                    load_idx(p, 2 * q + 1, ibB, jbB)
                    scatters(p, acc, ibA, rowsA)

                    @pl.when(q < n_pairs - 1)
                    def _():
                        load_idx(p, 2 * q + 2, ibA, jbA)

                    scatters(p, acc, ibB, rowsB)
                    return 0

                lax.fori_loop(0, n_pairs, pair3, 0)

            plsc.subcore_barrier()

            stages = (rowsA, rowsB)
            for z in range(rows_per_tile // ZROWS):
                r0 = tile_row0 + z * ZROWS
                stage = stages[z % 2].at[pl.ds(0, ZROWS), :]
                if z >= 2:
                    pltpu.make_async_copy(
                        stage, sp.at[c, p, pl.ds(r0, ZROWS), :], semF).wait()
                pltpu.sync_copy(acc.at[pl.ds(r0, ZROWS), :], stage)
                pltpu.async_copy(stage, sp.at[c, p, pl.ds(r0, ZROWS), :], semF)
            for z in range(2):
                r0 = tile_row0 + z * ZROWS
                stage = stages[z % 2].at[pl.ds(0, ZROWS), :]
                pltpu.make_async_copy(
                    stage, sp.at[c, p, pl.ds(r0, ZROWS), :], semF).wait()

        for p in range(4):
            run_pass(p, acc)

    return body


def _tc_combine(px, sp, Wi, Wj, Wff, block_n):
    n = px.shape[0]
    assert n % block_n == 0

    def body(px_ref, sp_ref, wi_ref, wj_ref, wff_ref, out_ref):
        a = jnp.dot(wi_ref[...], wff_ref[...], preferred_element_type=jnp.float32)
        b = jnp.dot(wj_ref[...], wff_ref[...], preferred_element_type=jnp.float32)
        deg = sp_ref[0, 3] + sp_ref[1, 3]
        for s in range(X):
            x = px_ref[:, s, :]
            ssum = sp_ref[0, s] + sp_ref[1, s]
            out_ref[:, s, :] = deg * jnp.dot(x, a, preferred_element_type=jnp.float32) \
                + jnp.dot(ssum, b, preferred_element_type=jnp.float32)

    return pl.pallas_call(
        body,
        grid=(n // block_n,),
        in_specs=[
            pl.BlockSpec((block_n, X, D), lambda i: (i, 0, 0)),
            pl.BlockSpec((2, 4, block_n, D), lambda i: (0, 0, i, 0)),
            pl.BlockSpec((D, D), lambda i: (0, 0)),
            pl.BlockSpec((D, D), lambda i: (0, 0)),
            pl.BlockSpec((D, D), lambda i: (0, 0)),
        ],
        out_specs=pl.BlockSpec((block_n, X, D), lambda i: (i, 0, 0)),
        out_shape=jax.ShapeDtypeStruct((n, X, D), jnp.float32),
    )(px, sp, Wi, Wj, Wff)


def kernel(ind_2, px, Wi, Wj, Wff):
    e = ind_2.shape[0]
    n = px.shape[0]

    ept = -(-e // (32 * 2 * CHUNK)) * (2 * CHUNK)
    e_pad = 32 * ept
    n_pad = -(-(n + 1) // (16 * ZROWS)) * (16 * ZROWS)

    ii = ind_2[:, 0]
    jj = ind_2[:, 1]
    ii = jnp.concatenate([ii, jnp.full((e_pad - e,), n, dtype=jnp.int32)])
    jj = jnp.concatenate([jj, jnp.zeros((e_pad - e,), dtype=jnp.int32)])
    ii2 = ii.reshape(e_pad // IDXW, IDXW)
    jj2 = jj.reshape(e_pad // IDXW, IDXW)

    px_t = jnp.transpose(px, (1, 0, 2))
    sc = _sc_edge_kernel(n_pad, e_pad)
    sp = sc(ii2, jj2, px_t[0], px_t[1], px_t[2])

    return _tc_combine(px, sp, Wi, Wj, Wff, block_n=2000)


# speedup vs baseline: 149.1441x; 1.2008x over previous
import functools

import jax
import jax.numpy as jnp
from jax import lax
from jax.experimental import pallas as pl
from jax.experimental.pallas import tpu as pltpu
from jax.experimental.pallas import tpu_sc as plsc

D = 16
X = 3
IDXW = 128
SUB = 4
CHUNK = SUB * IDXW
ZROWS = 448


def _sc_edge_kernel(n_pad, e_pad, n_nodes):
    n_tiles = 16
    rows_per_tile = n_pad // n_tiles
    ept = e_pad // n_tiles
    n_chunks = ept // CHUNK
    n_pairs = n_chunks // 2
    assert n_pad % (n_tiles * ZROWS) == 0
    assert ept % (2 * CHUNK) == 0 and CHUNK % IDXW == 0

    mesh = plsc.VectorSubcoreMesh(core_axis_name="c", subcore_axis_name="s")

    @functools.partial(
        pl.kernel,
        out_type=jax.ShapeDtypeStruct((4, n_pad, D), jnp.float32),
        mesh=mesh,
        compiler_params=pltpu.CompilerParams(use_tc_tiling_on_sc=False),
        scratch_types=[
            pltpu.VMEM((SUB, IDXW), jnp.int32),
            pltpu.VMEM((SUB, IDXW), jnp.int32),
            pltpu.VMEM((SUB, IDXW), jnp.int32),
            pltpu.VMEM((SUB, IDXW), jnp.int32),
            pltpu.VMEM((CHUNK, D), jnp.float32),
            pltpu.VMEM((CHUNK, D), jnp.float32),
            pltpu.VMEM_SHARED((n_pad, D), jnp.float32),
            pltpu.SemaphoreType.DMA,
            pltpu.SemaphoreType.DMA,
            pltpu.SemaphoreType.DMA,
        ],
    )
    def body(ii2, jj2, px_tf, sp,
             ibA, ibB, jbA, jbB, rowsA, rowsB, acc, semA, semB, semF):
        c = lax.axis_index("c")
        t = lax.axis_index("s")

        def fill_rows(ref, count, value):
            row = jnp.full((D,), value, dtype=jnp.float32)

            def fill(i, _):
                ref[i, :] = row
                return 0

            lax.fori_loop(0, count, fill, 0)

        tile_row0 = t * rows_per_tile
        idx_row0 = t * (ept // IDXW)

        def load_idx(k, ib, jb, with_j):
            r0 = idx_row0 + k * SUB
            pltpu.sync_copy(ii2.at[pl.ds(r0, SUB)], ib)
            if with_j:
                pltpu.sync_copy(jj2.at[pl.ds(r0, SUB)], jb)

        def zero_acc():
            fill_rows(rowsA, ZROWS, 0.0)
            zsrc = rowsA.at[pl.ds(0, ZROWS), :]
            for z in range(rows_per_tile // ZROWS):
                pltpu.sync_copy(
                    zsrc, acc.at[pl.ds(tile_row0 + z * ZROWS, ZROWS), :])

        def flush(out_idx):
            stages = (rowsA, rowsB)
            nz = rows_per_tile // ZROWS
            for z in range(nz):
                r0 = tile_row0 + z * ZROWS
                stage = stages[z % 2].at[pl.ds(0, ZROWS), :]
                if z >= 2:
                    pltpu.make_async_copy(
                        stage, sp.at[out_idx, pl.ds(r0, ZROWS), :], semF).wait()
                pltpu.sync_copy(acc.at[pl.ds(r0, ZROWS), :], stage)
                pltpu.async_copy(
                    stage, sp.at[out_idx, pl.ds(r0, ZROWS), :], semF)
            for z in range(nz - 2, nz):
                r0 = tile_row0 + z * ZROWS
                stage = stages[z % 2].at[pl.ds(0, ZROWS), :]
                pltpu.make_async_copy(
                    stage, sp.at[out_idx, pl.ds(r0, ZROWS), :], semF).wait()

        def slice_pass(tidx):
            table = px_tf.at[pl.ds(tidx * n_nodes, n_nodes), :]

            def gathers(jb, rows, sem):
                for r in range(SUB):
                    dst = rows.at[pl.ds(r * IDXW, IDXW), :]
                    pltpu.async_copy(table.at[jb.at[r]], dst, sem)

            def drain(jb, rows, sem):
                for r in range(SUB):
                    dst = rows.at[pl.ds(r * IDXW, IDXW), :]
                    pltpu.make_async_copy(table.at[jb.at[r]], dst, sem).wait()

            def scatters(ib, rows):
                for r in range(SUB):
                    src = rows.at[pl.ds(r * IDXW, IDXW), :]
                    pltpu.sync_copy(src, acc.at[ib.at[r]], add=True)

            load_idx(0, ibA, jbA, True)
            gathers(jbA, rowsA, semA)

            def pair(q, _):
                load_idx(2 * q + 1, ibB, jbB, True)
                gathers(jbB, rowsB, semB)
                drain(jbA, rowsA, semA)
                scatters(ibA, rowsA)

                @pl.when(q < n_pairs - 1)
                def _():
                    load_idx(2 * q + 2, ibA, jbA, True)
                    gathers(jbA, rowsA, semA)

                drain(jbB, rowsB, semB)
                scatters(ibB, rowsB)
                return 0

            lax.fori_loop(0, n_pairs, pair, 0)

        def deg_pass():
            ones_src = (rowsA.at[pl.ds(0, IDXW), :], rowsB.at[pl.ds(0, IDXW), :])

            def scatters(ib, src):
                for r in range(SUB):
                    pltpu.sync_copy(src, acc.at[ib.at[r]], add=True)

            load_idx(0, ibA, jbA, False)

            def pair3(q, _):
                load_idx(2 * q + 1, ibB, jbB, False)
                scatters(ibA, ones_src[0])

                @pl.when(q < n_pairs - 1)
                def _():
                    load_idx(2 * q + 2, ibA, jbA, False)

                scatters(ibB, ones_src[1])
                return 0

            lax.fori_loop(0, n_pairs, pair3, 0)

        zero_acc()
        plsc.subcore_barrier()
        slice_pass(2 * c)
        plsc.subcore_barrier()
        flush(2 * c)

        zero_acc()

        @pl.when(c == 1)
        def _():
            fill_rows(rowsA, IDXW, 1.0)
            fill_rows(rowsB, IDXW, 1.0)

        plsc.subcore_barrier()

        @pl.when(c == 0)
        def _():
            slice_pass(jnp.int32(1))

        @pl.when(c == 1)
        def _():
            deg_pass()

        plsc.subcore_barrier()
        flush(2 * c + 1)

    return body


def _tc_combine(px, sp, Wi, Wj, Wff, block_n):
    n = px.shape[0]
    assert n % block_n == 0

    def body(px_ref, sp_ref, wi_ref, wj_ref, wff_ref, out_ref):
        a = jnp.dot(wi_ref[...], wff_ref[...], preferred_element_type=jnp.float32)
        b = jnp.dot(wj_ref[...], wff_ref[...], preferred_element_type=jnp.float32)
        deg = sp_ref[3]
        for s in range(X):
            x = px_ref[:, s, :]
            ssum = sp_ref[s]
            out_ref[:, s, :] = deg * jnp.dot(x, a, preferred_element_type=jnp.float32) \
                + jnp.dot(ssum, b, preferred_element_type=jnp.float32)

    return pl.pallas_call(
        body,
        grid=(n // block_n,),
        in_specs=[
            pl.BlockSpec((block_n, X, D), lambda i: (i, 0, 0)),
            pl.BlockSpec((4, block_n, D), lambda i: (0, i, 0)),
            pl.BlockSpec((D, D), lambda i: (0, 0)),
            pl.BlockSpec((D, D), lambda i: (0, 0)),
            pl.BlockSpec((D, D), lambda i: (0, 0)),
        ],
        out_specs=pl.BlockSpec((block_n, X, D), lambda i: (i, 0, 0)),
        out_shape=jax.ShapeDtypeStruct((n, X, D), jnp.float32),
    )(px, sp, Wi, Wj, Wff)


def kernel(ind_2, px, Wi, Wj, Wff):
    e = ind_2.shape[0]
    n = px.shape[0]

    ept = -(-e // (16 * 2 * CHUNK)) * (2 * CHUNK)
    e_pad = 16 * ept
    n_pad = -(-(n + 1) // (16 * ZROWS)) * (16 * ZROWS)

    ii = jnp.concatenate([ind_2[:, 0], jnp.full((e_pad - e,), n, jnp.int32)])
    jj = jnp.concatenate([ind_2[:, 1], jnp.zeros((e_pad - e,), jnp.int32)])
    ii2 = ii.reshape(e_pad // IDXW, IDXW)
    jj2 = jj.reshape(e_pad // IDXW, IDXW)

    px_t = jnp.transpose(px, (1, 0, 2))
    px_tf = px_t.reshape(X * n, D)
    sc = _sc_edge_kernel(n_pad, e_pad, n)
    sp = sc(ii2, jj2, px_tf)

    return _tc_combine(px, sp, Wi, Wj, Wff, block_n=2000)


# speedup vs baseline: 163.9348x; 1.0992x over previous
import functools

import jax
import jax.numpy as jnp
from jax import lax
from jax.experimental import pallas as pl
from jax.experimental.pallas import tpu as pltpu
from jax.experimental.pallas import tpu_sc as plsc

D = 16
X = 3
IDXW = 128
SUB = 4
CHUNK = SUB * IDXW
ZROWS = 448


def _sc_edge_kernel(n_pad, e_pad, n_nodes):
    n_tiles = 16
    rows_per_tile = n_pad // n_tiles
    ept = e_pad // n_tiles
    n_chunks = ept // CHUNK
    n_pairs = n_chunks // 2
    assert n_pad % (n_tiles * ZROWS) == 0
    assert ept % (2 * CHUNK) == 0 and CHUNK % IDXW == 0

    mesh = plsc.VectorSubcoreMesh(core_axis_name="c", subcore_axis_name="s")

    @functools.partial(
        pl.kernel,
        out_type=jax.ShapeDtypeStruct((4, n_pad, D), jnp.float32),
        mesh=mesh,
        compiler_params=pltpu.CompilerParams(use_tc_tiling_on_sc=False),
        scratch_types=[
            pltpu.VMEM((SUB, IDXW), jnp.int32),
            pltpu.VMEM((SUB, IDXW), jnp.int32),
            pltpu.VMEM((SUB, IDXW), jnp.int32),
            pltpu.VMEM((SUB, IDXW), jnp.int32),
            pltpu.VMEM((CHUNK, D), jnp.float32),
            pltpu.VMEM((CHUNK, D), jnp.float32),
            pltpu.VMEM_SHARED((n_pad, D), jnp.float32),
            pltpu.SemaphoreType.DMA,
            pltpu.SemaphoreType.DMA,
            pltpu.SemaphoreType.DMA,
        ],
    )
    def body(ii2, jj2, px_tf, sp,
             ibA, ibB, jbA, jbB, rowsA, rowsB, acc, semA, semB, semF):
        c = lax.axis_index("c")
        t = lax.axis_index("s")

        def fill_rows(ref, count, value):
            row = jnp.full((D,), value, dtype=jnp.float32)

            def fill(i, _):
                ref[i, :] = row
                return 0

            lax.fori_loop(0, count, fill, 0)

        tile_row0 = t * rows_per_tile
        idx_row0 = t * (ept // IDXW)

        def load_idx(k, ib, jb, with_j):
            r0 = idx_row0 + k * SUB
            pltpu.sync_copy(ii2.at[pl.ds(r0, SUB)], ib)
            if with_j:
                pltpu.sync_copy(jj2.at[pl.ds(r0, SUB)], jb)

        def zero_acc():
            fill_rows(rowsA, ZROWS, 0.0)
            zsrc = rowsA.at[pl.ds(0, ZROWS), :]
            for z in range(rows_per_tile // ZROWS):
                pltpu.sync_copy(
                    zsrc, acc.at[pl.ds(tile_row0 + z * ZROWS, ZROWS), :])

        def flush(out_idx):
            stages = (rowsA, rowsB)
            nz = rows_per_tile // ZROWS
            for z in range(nz):
                r0 = tile_row0 + z * ZROWS
                stage = stages[z % 2].at[pl.ds(0, ZROWS), :]
                if z >= 2:
                    pltpu.make_async_copy(
                        stage, sp.at[out_idx, pl.ds(r0, ZROWS), :], semF).wait()
                pltpu.sync_copy(acc.at[pl.ds(r0, ZROWS), :], stage)
                pltpu.async_copy(
                    stage, sp.at[out_idx, pl.ds(r0, ZROWS), :], semF)
            for z in range(nz - 2, nz):
                r0 = tile_row0 + z * ZROWS
                stage = stages[z % 2].at[pl.ds(0, ZROWS), :]
                pltpu.make_async_copy(
                    stage, sp.at[out_idx, pl.ds(r0, ZROWS), :], semF).wait()

        def slice_pass(tidx):
            table = px_tf.at[pl.ds(tidx * n_nodes, n_nodes), :]

            def gathers(jb, rows, sem):
                for r in range(SUB):
                    dst = rows.at[pl.ds(r * IDXW, IDXW), :]
                    pltpu.async_copy(table.at[jb.at[r]], dst, sem)

            def drain(jb, rows, sem):
                for r in range(SUB):
                    dst = rows.at[pl.ds(r * IDXW, IDXW), :]
                    pltpu.make_async_copy(table.at[jb.at[r]], dst, sem).wait()

            def scatters(ib, rows):
                for r in range(SUB):
                    src = rows.at[pl.ds(r * IDXW, IDXW), :]
                    pltpu.sync_copy(src, acc.at[ib.at[r]], add=True)

            load_idx(0, ibA, jbA, True)
            gathers(jbA, rowsA, semA)

            def pair(q, _):
                load_idx(2 * q + 1, ibB, jbB, True)
                gathers(jbB, rowsB, semB)
                drain(jbA, rowsA, semA)
                scatters(ibA, rowsA)

                @pl.when(q < n_pairs - 1)
                def _():
                    load_idx(2 * q + 2, ibA, jbA, True)
                    gathers(jbA, rowsA, semA)

                drain(jbB, rowsB, semB)
                scatters(ibB, rowsB)
                return 0

            lax.fori_loop(0, n_pairs, pair, 0)

        def deg_pass():
            ones_src = (rowsA.at[pl.ds(0, IDXW), :], rowsB.at[pl.ds(0, IDXW), :])

            def scatters(ib, src):
                for r in range(SUB):
                    pltpu.sync_copy(src, acc.at[ib.at[r]], add=True)

            load_idx(0, ibA, jbA, False)

            def pair3(q, _):
                load_idx(2 * q + 1, ibB, jbB, False)
                scatters(ibA, ones_src[0])

                @pl.when(q < n_pairs - 1)
                def _():
                    load_idx(2 * q + 2, ibA, jbA, False)

                scatters(ibB, ones_src[1])
                return 0

            lax.fori_loop(0, n_pairs, pair3, 0)

        zero_acc()
        plsc.subcore_barrier()
        slice_pass(2 * c)
        plsc.subcore_barrier()
        flush(2 * c)

        zero_acc()

        @pl.when(c == 1)
        def _():
            fill_rows(rowsA, IDXW, 1.0)
            fill_rows(rowsB, IDXW, 1.0)

        plsc.subcore_barrier()

        @pl.when(c == 0)
        def _():
            slice_pass(jnp.int32(1))

        @pl.when(c == 1)
        def _():
            deg_pass()

        plsc.subcore_barrier()
        flush(2 * c + 1)

    return body


def _tc_combine(px_tf, sp, Wi, Wj, Wff, n, n_pad, block_r):
    rows_per_plane = n_pad * D // 128
    assert rows_per_plane % block_r == 0
    nb = rows_per_plane // block_r
    px_v = px_tf.reshape(X * rows_per_plane, 128)
    sp_v = sp.reshape(4, rows_per_plane, 128)

    def body(px_ref, sps_ref, spd_ref, wi_ref, wj_ref, wff_ref, out_ref):
        eye8 = jnp.eye(8, dtype=jnp.float32)
        a = jnp.dot(wi_ref[...], wff_ref[...], preferred_element_type=jnp.float32)
        b = jnp.dot(wj_ref[...], wff_ref[...], preferred_element_type=jnp.float32)
        ka = jnp.kron(eye8, a)
        kb = jnp.kron(eye8, b)
        out_ref[...] = spd_ref[0] * jnp.dot(
            px_ref[...], ka, preferred_element_type=jnp.float32
        ) + jnp.dot(sps_ref[0], kb, preferred_element_type=jnp.float32)

    outf = pl.pallas_call(
        body,
        grid=(X, nb),
        in_specs=[
            pl.BlockSpec((block_r, 128), lambda p, i: (p * nb + i, 0)),
            pl.BlockSpec((1, block_r, 128), lambda p, i: (p, i, 0)),
            pl.BlockSpec((1, block_r, 128), lambda p, i: (3, i, 0)),
            pl.BlockSpec((D, D), lambda p, i: (0, 0)),
            pl.BlockSpec((D, D), lambda p, i: (0, 0)),
            pl.BlockSpec((D, D), lambda p, i: (0, 0)),
        ],
        out_specs=pl.BlockSpec((block_r, 128), lambda p, i: (p * nb + i, 0)),
        out_shape=jax.ShapeDtypeStruct((X * rows_per_plane, 128), jnp.float32),
    )(px_v, sp_v, sp_v, Wi, Wj, Wff)

    return jnp.transpose(outf.reshape(X, n_pad, D)[:, :n, :], (1, 0, 2))


def kernel(ind_2, px, Wi, Wj, Wff):
    e = ind_2.shape[0]
    n = px.shape[0]

    ept = -(-e // (16 * 2 * CHUNK)) * (2 * CHUNK)
    e_pad = 16 * ept
    n_pad = -(-(n + 1) // (16 * ZROWS)) * (16 * ZROWS)

    assert e % IDXW == 0
    pad_rows = (e_pad - e) // IDXW
    ii2 = jnp.concatenate(
        [ind_2[:, 0].reshape(e // IDXW, IDXW),
         jnp.full((pad_rows, IDXW), n, jnp.int32)])
    jj2 = jnp.concatenate(
        [ind_2[:, 1].reshape(e // IDXW, IDXW),
         jnp.zeros((pad_rows, IDXW), jnp.int32)])

    px_t = lax.pad(jnp.transpose(px, (1, 0, 2)),
                   jnp.float32(0), ((0, 0, 0), (0, n_pad - n, 0), (0, 0, 0)))
    px_tf = px_t.reshape(X * n_pad, D)
    sc = _sc_edge_kernel(n_pad, e_pad, n_pad)
    sp = sc(ii2, jj2, px_tf)

    return _tc_combine(px_tf, sp, Wi, Wj, Wff, n, n_pad, block_r=1568)
